# Initial kernel scaffold; baseline (speedup 1.0000x reference)
import jax
import jax.numpy as jnp
from jax.experimental import pallas as pl


def kernel(x, adj, w_gnn, b_gnn, w_upd, b_upd, w_gate, b_gate):
    raise NotImplementedError("write your pallas kernel here")


# speedup vs baseline: 1.1448x; 1.1448x over previous
import jax
import jax.numpy as jnp
from jax.experimental import pallas as pl
from jax.experimental.pallas import tpu as pltpu


def _round_up(v, m):
    return ((v + m - 1) // m) * m


def _pad2(a, rows, cols):
    r, c = a.shape
    if r == rows and c == cols:
        return a
    return jnp.pad(a, ((0, rows - r), (0, cols - c)))


def _proj_kernel(x_ref, w_ref, m_ref):
    m_ref[...] = jnp.dot(x_ref[...].astype(w_ref.dtype), w_ref[...],
                         preferred_element_type=jnp.float32).astype(m_ref.dtype)


def _fused_kernel(adj_ref, x_ref, m_ref, w_ug_ref, w_gu_ref, b_ref, out_ref):
    hp = out_ref.shape[-1]
    acc = jnp.dot(adj_ref[...], m_ref[...], preferred_element_type=jnp.float32)

    b = b_ref[...]
    z = jnp.maximum(acc + b[:, :hp], 0.0)

    x = x_ref[...]
    ug = jnp.dot(x.astype(w_ug_ref.dtype), w_ug_ref[...],
                 preferred_element_type=jnp.float32)
    u = ug[:, :hp] + b[:, hp:2 * hp] + z
    gate_pre = jnp.dot(u.astype(w_gu_ref.dtype), w_gu_ref[...],
                       preferred_element_type=jnp.float32)
    gate_pre = gate_pre + ug[:, hp:] + b[:, 2 * hp:]

    g = jax.nn.sigmoid(gate_pre)
    xf = x.astype(jnp.float32)
    out_ref[...] = (jnp.tanh(u) * g + xf * (1.0 - g)).astype(out_ref.dtype)


def kernel(x, adj, w_gnn, b_gnn, w_upd, b_upd, w_gate, b_gate):
    mm_dtype = jnp.bfloat16
    N, H = x.shape
    Hp = _round_up(H, 128)
    Np = _round_up(N, 128)
    item = jnp.dtype(mm_dtype).itemsize

    TM = 512
    while Np % TM:
        TM //= 2

    x_p = _pad2(x.astype(jnp.float32), Np, Hp)
    if adj.shape == (Np, Np) and adj.dtype == jnp.dtype(mm_dtype):
        adj_p = adj
    else:
        adj_p = _pad2(adj, Np, Np).astype(mm_dtype)

    w_gnn_p = _pad2(w_gnn, Hp, Hp).astype(mm_dtype)
    wg_u = w_gate[:H, :]
    wg_x = w_gate[H:, :]
    w_ug = jnp.concatenate([_pad2(w_upd, Hp, Hp), _pad2(wg_x, Hp, Hp)],
                           axis=1).astype(mm_dtype)
    w_gu = _pad2(wg_u, Hp, Hp).astype(mm_dtype)
    b_cat = jnp.concatenate(
        [jnp.pad(b_gnn.astype(jnp.float32), (0, Hp - H)),
         jnp.pad(b_upd.astype(jnp.float32), (0, Hp - H)),
         jnp.pad(b_gate.astype(jnp.float32), (0, Hp - H))]).reshape(1, 3 * Hp)

    m = pl.pallas_call(
        _proj_kernel,
        out_shape=jax.ShapeDtypeStruct((Np, Hp), mm_dtype),
        grid=(Np // TM,),
        in_specs=[pl.BlockSpec((TM, Hp), lambda i: (i, 0)),
                  pl.BlockSpec((Hp, Hp), lambda i: (0, 0),
                               pipeline_mode=pl.Buffered(1))],
        out_specs=pl.BlockSpec((TM, Hp), lambda i: (i, 0)),
        compiler_params=pltpu.CompilerParams(
            dimension_semantics=("parallel",)),
    )(x_p, w_gnn_p)

    vmem_bytes = (2 * TM * Np * item
                  + Np * Hp * item
                  + 2 * TM * Hp * 4
                  + Hp * 2 * Hp * item
                  + Hp * Hp * item
                  + 3 * Hp * 4
                  + 2 * TM * Hp * 4)
    vmem_limit = int(min(max(2 * vmem_bytes, 32 << 20), 56 << 20))

    flops = 2 * Np * Np * Hp + 6 * Np * Hp * Hp
    bytes_accessed = (Np * Np * item + Np * Hp * item + Np * Hp * 4
                      + 3 * Hp * Hp * item + Np * Hp * 4)
    cost = pl.CostEstimate(flops=flops, transcendentals=2 * Np * Hp,
                           bytes_accessed=bytes_accessed)

    out_p = pl.pallas_call(
        _fused_kernel,
        out_shape=jax.ShapeDtypeStruct((Np, Hp), x.dtype),
        grid=(Np // TM,),
        in_specs=[pl.BlockSpec((TM, Np), lambda i: (i, 0)),
                  pl.BlockSpec((TM, Hp), lambda i: (i, 0)),
                  pl.BlockSpec((Np, Hp), lambda i: (0, 0),
                               pipeline_mode=pl.Buffered(1)),
                  pl.BlockSpec((Hp, 2 * Hp), lambda i: (0, 0),
                               pipeline_mode=pl.Buffered(1)),
                  pl.BlockSpec((Hp, Hp), lambda i: (0, 0),
                               pipeline_mode=pl.Buffered(1)),
                  pl.BlockSpec((1, 3 * Hp), lambda i: (0, 0),
                               pipeline_mode=pl.Buffered(1))],
        out_specs=pl.BlockSpec((TM, Hp), lambda i: (i, 0)),
        compiler_params=pltpu.CompilerParams(
            dimension_semantics=("parallel",),
            vmem_limit_bytes=vmem_limit),
        cost_estimate=cost,
    )(adj_p, x_p, m, w_ug, w_gu, b_cat)

    return out_p[:N, :H]


# speedup vs baseline: 1.1460x; 1.0010x over previous
import jax
import jax.numpy as jnp
from jax.experimental import pallas as pl
from jax.experimental.pallas import tpu as pltpu


def _gnn_kernel(adj_ref, x_ref, w_gnn_ref, w_upd_ref, w_gate_ref,
                bg_ref, bu_ref, bga_ref, out_ref, acc_ref, xb_ref):
    c = pl.program_id(0)
    j = pl.program_id(1)
    k = pl.program_id(2)
    nj = pl.num_programs(1)
    nk = pl.num_programs(2)
    tk = x_ref.shape[0]
    tm = out_ref.shape[0]
    hp = out_ref.shape[1]

    @pl.when(j == 0)
    def _():
        xb_ref[pl.ds(k * tk, tk), :] = x_ref[...].astype(xb_ref.dtype)

    t = jnp.dot(adj_ref[...], xb_ref[pl.ds(k * tk, tk), :],
                preferred_element_type=jnp.float32)

    @pl.when(k == 0)
    def _():
        acc_ref[...] = t

    @pl.when(k > 0)
    def _():
        acc_ref[...] += t

    @pl.when(k == nk - 1)
    def _():
        bf = jnp.bfloat16
        wg = w_gnn_ref[...].astype(bf)
        wu = w_upd_ref[...].astype(bf)
        wgu = w_gate_ref[:hp, :].astype(bf)
        wgx = w_gate_ref[hp:, :].astype(bf)
        bg = bg_ref[...]
        bu = bu_ref[...]
        bga = bga_ref[...]

        ch = 512 if tm % 512 == 0 else tm
        for ci in range(tm // ch):
            rows = pl.ds(ci * ch, ch)
            grow = (c * nj + j) * tm + ci * ch
            t16 = acc_ref[rows, :].astype(bf)
            z = jnp.maximum(
                jnp.dot(t16, wg, preferred_element_type=jnp.float32) + bg,
                0.0)
            x16 = xb_ref[pl.ds(grow, ch), :]
            u = jnp.dot(x16, wu, preferred_element_type=jnp.float32) + bu + z
            gate_pre = (jnp.dot(u.astype(bf), wgu,
                                preferred_element_type=jnp.float32)
                        + jnp.dot(x16, wgx,
                                  preferred_element_type=jnp.float32)
                        + bga)
            g = jax.nn.sigmoid(gate_pre)
            xf = x16.astype(jnp.float32)
            out_ref[rows, :] = (jnp.tanh(u) * g + xf * (1.0 - g)
                                ).astype(out_ref.dtype)


def _round_up(v, m):
    return ((v + m - 1) // m) * m


def _pad2(a, rows, cols):
    r, c = a.shape
    if r == rows and c == cols:
        return a
    return jnp.pad(a, ((0, rows - r), (0, cols - c)))


def kernel(x, adj, w_gnn, b_gnn, w_upd, b_upd, w_gate, b_gate):
    mm_dtype = jnp.bfloat16
    N, H = x.shape
    Hp = _round_up(H, 128)
    Np = _round_up(N, 128)
    item = jnp.dtype(mm_dtype).itemsize

    NC = 2 if Np % 256 == 0 else 1
    NJ = 2 if Np % (NC * 256) == 0 else 1
    TM = Np // (NC * NJ)
    TK = 1024
    while Np % TK:
        TK //= 2
    NK = Np // TK

    x_p = _pad2(x.astype(jnp.float32), Np, Hp)
    if adj.shape == (Np, Np) and adj.dtype == jnp.dtype(mm_dtype):
        adj_p = adj
    else:
        adj_p = _pad2(adj, Np, Np).astype(mm_dtype)
    w_gnn_p = _pad2(w_gnn, Hp, Hp)
    w_upd_p = _pad2(w_upd, Hp, Hp)
    if H == Hp:
        w_gate_p = w_gate
    else:
        w_gate_p = jnp.concatenate([_pad2(w_gate[:H], Hp, Hp),
                                    _pad2(w_gate[H:], Hp, Hp)], axis=0)
    bg = jnp.pad(b_gnn.astype(jnp.float32), (0, Hp - H)).reshape(1, Hp)
    bu = jnp.pad(b_upd.astype(jnp.float32), (0, Hp - H)).reshape(1, Hp)
    bga = jnp.pad(b_gate.astype(jnp.float32), (0, Hp - H)).reshape(1, Hp)

    vmem_limit = int(48 << 20)
    flops = 2 * Np * Np * Hp + 8 * Np * Hp * Hp
    bytes_accessed = (Np * Np * item + Np * Hp * 4 * (NC + 1)
                      + 4 * Hp * Hp * 4)
    cost = pl.CostEstimate(flops=flops, transcendentals=2 * Np * Hp,
                           bytes_accessed=bytes_accessed)

    b1 = pl.Buffered(1)
    out_p = pl.pallas_call(
        _gnn_kernel,
        out_shape=jax.ShapeDtypeStruct((Np, Hp), x.dtype),
        grid=(NC, NJ, NK),
        in_specs=[
            pl.BlockSpec((TM, TK), lambda c, j, k: (c * NJ + j, k)),
            pl.BlockSpec((TK, Hp),
                         lambda c, j, k: (jnp.where(j == 0, k, 0), 0)),
            pl.BlockSpec((Hp, Hp), lambda c, j, k: (0, 0), pipeline_mode=b1),
            pl.BlockSpec((Hp, Hp), lambda c, j, k: (0, 0), pipeline_mode=b1),
            pl.BlockSpec((2 * Hp, Hp), lambda c, j, k: (0, 0),
                         pipeline_mode=b1),
            pl.BlockSpec((1, Hp), lambda c, j, k: (0, 0), pipeline_mode=b1),
            pl.BlockSpec((1, Hp), lambda c, j, k: (0, 0), pipeline_mode=b1),
            pl.BlockSpec((1, Hp), lambda c, j, k: (0, 0), pipeline_mode=b1),
        ],
        out_specs=pl.BlockSpec((TM, Hp), lambda c, j, k: (c * NJ + j, 0)),
        scratch_shapes=[pltpu.VMEM((TM, Hp), jnp.float32),
                        pltpu.VMEM((Np, Hp), mm_dtype)],
        compiler_params=pltpu.CompilerParams(
            dimension_semantics=("parallel", "arbitrary", "arbitrary"),
            vmem_limit_bytes=vmem_limit),
        cost_estimate=cost,
    )(adj_p, x_p, w_gnn_p, w_upd_p, w_gate_p, bg, bu, bga)

    return out_p[:N, :H]


# speedup vs baseline: 1.3200x; 1.1518x over previous
import jax
import jax.numpy as jnp
from jax.experimental import pallas as pl
from jax.experimental.pallas import tpu as pltpu


def _gnn_kernel(adj_ref, x_ref, w_gnn_ref, w_upd_ref, w_gate_ref,
                bg_ref, bu_ref, bga_ref, out_ref, m_ref):
    c = pl.program_id(0)
    j = pl.program_id(1)
    nj = pl.num_programs(1)
    tm = out_ref.shape[0]
    hp = out_ref.shape[1]
    np_ = m_ref.shape[0]
    bf = jnp.bfloat16

    @pl.when(j == 0)
    def _():
        wg = w_gnn_ref[...].astype(bf)
        mch = 1024
        while np_ % mch:
            mch //= 2
        for mi in range(np_ // mch):
            rows = pl.ds(mi * mch, mch)
            m_ref[rows, :] = jnp.dot(
                x_ref[rows, :].astype(bf), wg,
                preferred_element_type=jnp.float32).astype(bf)

    acc = jnp.dot(adj_ref[...], m_ref[...], preferred_element_type=jnp.float32)

    row0 = (c * nj + j) * tm
    z = jnp.maximum(acc + bg_ref[...], 0.0)
    xf = x_ref[pl.ds(row0, tm), :]
    x16 = xf.astype(bf)
    u = jnp.dot(x16, w_upd_ref[...].astype(bf),
                preferred_element_type=jnp.float32) + bu_ref[...] + z
    gate_pre = (jnp.dot(u.astype(bf), w_gate_ref[:hp, :].astype(bf),
                        preferred_element_type=jnp.float32)
                + jnp.dot(x16, w_gate_ref[hp:, :].astype(bf),
                          preferred_element_type=jnp.float32)
                + bga_ref[...])
    g = jax.nn.sigmoid(gate_pre)
    out_ref[...] = (jnp.tanh(u) * g + xf * (1.0 - g)).astype(out_ref.dtype)


def _round_up(v, m):
    return ((v + m - 1) // m) * m


def _pad2(a, rows, cols):
    r, c = a.shape
    if r == rows and c == cols:
        return a
    return jnp.pad(a, ((0, rows - r), (0, cols - c)))


def kernel(x, adj, w_gnn, b_gnn, w_upd, b_upd, w_gate, b_gate):
    mm_dtype = jnp.bfloat16
    N, H = x.shape
    Hp = _round_up(H, 128)
    Np = _round_up(N, 128)
    item = jnp.dtype(mm_dtype).itemsize

    TM = 512
    while Np % TM:
        TM //= 2
    NC = 2 if (Np // TM) % 2 == 0 else 1
    NJ = Np // TM // NC

    x_p = _pad2(x.astype(jnp.float32), Np, Hp)
    if adj.shape == (Np, Np) and adj.dtype == jnp.dtype(mm_dtype):
        adj_p = adj
    else:
        adj_p = _pad2(adj, Np, Np).astype(mm_dtype)
    w_gnn_p = _pad2(w_gnn, Hp, Hp)
    w_upd_p = _pad2(w_upd, Hp, Hp)
    if H == Hp:
        w_gate_p = w_gate
    else:
        w_gate_p = jnp.concatenate([_pad2(w_gate[:H], Hp, Hp),
                                    _pad2(w_gate[H:], Hp, Hp)], axis=0)
    bg = jnp.pad(b_gnn.astype(jnp.float32), (0, Hp - H)).reshape(1, Hp)
    bu = jnp.pad(b_upd.astype(jnp.float32), (0, Hp - H)).reshape(1, Hp)
    bga = jnp.pad(b_gate.astype(jnp.float32), (0, Hp - H)).reshape(1, Hp)

    vmem_limit = int(48 << 20)
    flops = 2 * Np * Np * Hp + 8 * Np * Hp * Hp
    bytes_accessed = (Np * Np * item + Np * Hp * 4 * (NC + 1)
                      + 4 * Hp * Hp * 4)
    cost = pl.CostEstimate(flops=flops, transcendentals=2 * Np * Hp,
                           bytes_accessed=bytes_accessed)

    b1 = pl.Buffered(1)
    out_p = pl.pallas_call(
        _gnn_kernel,
        out_shape=jax.ShapeDtypeStruct((Np, Hp), x.dtype),
        grid=(NC, NJ),
        in_specs=[
            pl.BlockSpec((TM, Np), lambda c, j: (c * NJ + j, 0)),
            pl.BlockSpec((Np, Hp), lambda c, j: (0, 0),
                         pipeline_mode=b1),
            pl.BlockSpec((Hp, Hp), lambda c, j: (0, 0), pipeline_mode=b1),
            pl.BlockSpec((Hp, Hp), lambda c, j: (0, 0), pipeline_mode=b1),
            pl.BlockSpec((2 * Hp, Hp), lambda c, j: (0, 0), pipeline_mode=b1),
            pl.BlockSpec((1, Hp), lambda c, j: (0, 0), pipeline_mode=b1),
            pl.BlockSpec((1, Hp), lambda c, j: (0, 0), pipeline_mode=b1),
            pl.BlockSpec((1, Hp), lambda c, j: (0, 0), pipeline_mode=b1),
        ],
        out_specs=pl.BlockSpec((TM, Hp), lambda c, j: (c * NJ + j, 0)),
        scratch_shapes=[pltpu.VMEM((Np, Hp), mm_dtype)],
        compiler_params=pltpu.CompilerParams(
            dimension_semantics=("parallel", "arbitrary"),
            vmem_limit_bytes=vmem_limit),
        cost_estimate=cost,
    )(adj_p, x_p, w_gnn_p, w_upd_p, w_gate_p, bg, bu, bga)

    return out_p[:N, :H]
